# baseline (speedup 1.0000x reference)
import functools

import jax
import jax.numpy as jnp
from jax.experimental import pallas as pl

N = 8192
D = 5
BM = 256


def _gauss_body(refT_ref, refblk_ref, u_ref, out_ref):
    refT = refT_ref[...]
    a = refblk_ref[...]

    nsq = -jnp.sum(refT * refT, axis=0, keepdims=True)
    refT2 = refT + refT

    acc = jnp.broadcast_to(nsq, (BM, N))
    for d in range(D):
        acc = acc + a[:, d][:, None] * refT2[d, :][None, :]

    w = jnp.exp(acc).astype(jnp.bfloat16)
    ub = u_ref[...].astype(jnp.bfloat16)
    o = jnp.dot(w, ub, preferred_element_type=jnp.float32)

    scale = jnp.exp(-jnp.sum(a * a, axis=1, keepdims=True))
    out_ref[...] = o * scale


@jax.jit
def kernel(U, ref):
    n, d = ref.shape
    l = U.shape[1]
    refT = ref.T

    grid = (n // BM,)
    out = pl.pallas_call(
        _gauss_body,
        grid=grid,
        in_specs=[
            pl.BlockSpec((d, n), lambda i: (0, 0)),
            pl.BlockSpec((BM, d), lambda i: (i, 0)),
            pl.BlockSpec((n, l), lambda i: (0, 0)),
        ],
        out_specs=pl.BlockSpec((BM, l), lambda i: (i, 0)),
        out_shape=jax.ShapeDtypeStruct((n, l), jnp.float32),
    )(refT, ref, U)
    return out


# speedup vs baseline: 1.1559x; 1.1559x over previous
import functools
import math

import jax
import jax.numpy as jnp
from jax.experimental import pallas as pl
from jax.experimental.pallas import tpu as pltpu

N = 8192
D = 5
BM = 256

_LOG2E = math.log2(math.e)


def _gauss_body(refT_ref, refblk_ref, u_ref, out_ref, us_ref):
    i = pl.program_id(0)

    @pl.when(i == 0)
    def _init_us():
        u = u_ref[...]
        r = refT_ref[...]
        sq = jnp.sum(r * r, axis=0, keepdims=True)
        ex = jnp.exp2((-_LOG2E) * sq)
        exc = jnp.transpose(ex, (1, 0))
        us_ref[...] = (u * exc).astype(jnp.bfloat16)

    a = refblk_ref[...]
    a2 = a * (2.0 * _LOG2E)
    refT = refT_ref[...]

    acc = a2[:, 0][:, None] * refT[0, :][None, :]
    for d in range(1, D):
        acc = acc + a2[:, d][:, None] * refT[d, :][None, :]

    w = jnp.exp2(acc).astype(jnp.bfloat16)
    o = jnp.dot(w, us_ref[...], preferred_element_type=jnp.float32)

    scale = jnp.exp2((-_LOG2E) * jnp.sum(a * a, axis=1, keepdims=True))
    out_ref[...] = o * scale


@jax.jit
def kernel(U, ref):
    n, d = ref.shape
    l = U.shape[1]
    refT = ref.T

    grid = (n // BM,)
    out = pl.pallas_call(
        _gauss_body,
        grid=grid,
        in_specs=[
            pl.BlockSpec((d, n), lambda i: (0, 0)),
            pl.BlockSpec((BM, d), lambda i: (i, 0)),
            pl.BlockSpec((n, l), lambda i: (0, 0)),
        ],
        out_specs=pl.BlockSpec((BM, l), lambda i: (i, 0)),
        out_shape=jax.ShapeDtypeStruct((n, l), jnp.float32),
        scratch_shapes=[pltpu.VMEM((n, l), jnp.bfloat16)],
    )(refT, ref, U)
    return out


# speedup vs baseline: 1.1624x; 1.0056x over previous
import functools
import math

import jax
import jax.numpy as jnp
from jax.experimental import pallas as pl
from jax.experimental.pallas import tpu as pltpu

N = 8192
D = 5
BM = 256

_LOG2E = math.log2(math.e)


def _gauss_body(refT_ref, refblk_ref, u_ref, out_ref, us_ref):
    i = pl.program_id(0)

    @pl.when(i == 0)
    def _init_us():
        u = u_ref[...]
        r = refT_ref[...]
        sq = jnp.sum(r * r, axis=0, keepdims=True)
        ex = jnp.exp2((-_LOG2E) * sq)
        exc = jnp.transpose(ex, (1, 0))
        us_ref[...] = (u * exc).astype(jnp.bfloat16)

    a = refblk_ref[...]
    a2 = a * (2.0 * _LOG2E)

    CH = N // 4
    o = jnp.zeros((BM, u_ref.shape[1]), jnp.float32)
    for c in range(0, N, CH):
        bt = refT_ref[:, c:c + CH]
        acc = a2[:, 0][:, None] * bt[0, :][None, :]
        for d in range(1, D):
            acc = acc + a2[:, d][:, None] * bt[d, :][None, :]
        w = jnp.exp2(acc).astype(jnp.bfloat16)
        o = o + jnp.dot(w, us_ref[c:c + CH, :],
                        preferred_element_type=jnp.float32)

    scale = jnp.exp2((-_LOG2E) * jnp.sum(a * a, axis=1, keepdims=True))
    out_ref[...] = o * scale


@jax.jit
def kernel(U, ref):
    n, d = ref.shape
    l = U.shape[1]
    refT = ref.T

    grid = (n // BM,)
    out = pl.pallas_call(
        _gauss_body,
        grid=grid,
        in_specs=[
            pl.BlockSpec((d, n), lambda i: (0, 0)),
            pl.BlockSpec((BM, d), lambda i: (i, 0)),
            pl.BlockSpec((n, l), lambda i: (0, 0)),
        ],
        out_specs=pl.BlockSpec((BM, l), lambda i: (i, 0)),
        out_shape=jax.ShapeDtypeStruct((n, l), jnp.float32),
        scratch_shapes=[pltpu.VMEM((n, l), jnp.bfloat16)],
    )(refT, ref, U)
    return out
